# Initial kernel scaffold; baseline (speedup 1.0000x reference)
import jax
import jax.numpy as jnp
from jax.experimental import pallas as pl


def kernel(world_pos, prev_world_pos, mesh_pos, node_type, cells, params, is_training):
    raise NotImplementedError("write your pallas kernel here")


# speedup vs baseline: 2.3973x; 2.3973x over previous
import functools

import jax
import jax.numpy as jnp
from jax import lax
from jax.experimental import pallas as pl
from jax.experimental.pallas import tpu as pltpu
from jax.experimental.pallas import tpu_sc as plsc

NTYPE = 9
D = 128
NW = 32
CH = 128
TE = 1024
F32 = jnp.float32


def _cdiv(a, b):
    return (a + b - 1) // b


@functools.lru_cache(maxsize=None)
def _gather2_kernel(nrows, width, epad):
    mesh = plsc.VectorSubcoreMesh(core_axis_name="c", subcore_axis_name="s")
    bpw = epad // NW
    nch = bpw // CH

    @functools.partial(
        pl.kernel,
        out_type=[jax.ShapeDtypeStruct((epad, width), F32)] * 2,
        mesh=mesh,
        scratch_types=[
            pltpu.VMEM((CH,), jnp.int32),
            pltpu.VMEM((CH,), jnp.int32),
            pltpu.VMEM((CH, width), F32),
            pltpu.VMEM((CH, width), F32),
            pltpu.SemaphoreType.DMA,
            pltpu.SemaphoreType.DMA,
        ],
    )
    def k(t1_hbm, t2_hbm, i1_hbm, i2_hbm, g1_hbm, g2_hbm,
          i1_v, i2_v, r1_v, r2_v, s1, s2):
        wid = lax.axis_index("s") * 2 + lax.axis_index("c")
        base = wid * bpw

        def body(i, carry):
            off = base + i * CH
            pltpu.sync_copy(i1_hbm.at[pl.ds(off, CH)], i1_v)
            pltpu.sync_copy(i2_hbm.at[pl.ds(off, CH)], i2_v)
            c1 = pltpu.async_copy(t1_hbm.at[i1_v], r1_v, s1)
            c2 = pltpu.async_copy(t2_hbm.at[i2_v], r2_v, s2)
            c1.wait()
            c2.wait()
            pltpu.sync_copy(r1_v, g1_hbm.at[pl.ds(off, CH)])
            pltpu.sync_copy(r2_v, g2_hbm.at[pl.ds(off, CH)])
            return carry

        lax.fori_loop(0, nch, body, 0)

    return k


def _sc_gather2(t1, t2, i1, i2):
    nrows, width = t1.shape
    return _gather2_kernel(nrows, width, i1.shape[0])(t1, t2, i1, i2)


@functools.lru_cache(maxsize=None)
def _scatter_kernel(npad, epad):
    mesh = plsc.VectorSubcoreMesh(core_axis_name="c", subcore_axis_name="s")
    bpw = epad // NW
    nch = bpw // CH
    stripe = npad // 16

    @functools.partial(
        pl.kernel,
        out_type=jax.ShapeDtypeStruct((2 * npad, D), F32),
        mesh=mesh,
        scratch_types=[
            pltpu.VMEM((CH,), jnp.int32),
            pltpu.VMEM((CH, D), F32),
            pltpu.VMEM_SHARED((npad, D), F32),
            pltpu.SemaphoreType.DMA,
        ],
    )
    def k(ne_hbm, ri_hbm, z_hbm, out_hbm, ri_v, rows_v, acc_sh, sem):
        cid = lax.axis_index("c")
        sid = lax.axis_index("s")
        wid = sid * 2 + cid
        pltpu.sync_copy(z_hbm, acc_sh.at[pl.ds(sid * stripe, stripe)])
        plsc.subcore_barrier()
        base = wid * bpw

        def body(i, carry):
            off = base + i * CH
            pltpu.sync_copy(ri_hbm.at[pl.ds(off, CH)], ri_v)
            pltpu.sync_copy(ne_hbm.at[pl.ds(off, CH)], rows_v)
            pltpu.sync_copy(rows_v, acc_sh.at[ri_v], add=True)
            return carry

        lax.fori_loop(0, nch, body, 0)
        plsc.subcore_barrier()
        r0 = sid * stripe
        pltpu.sync_copy(acc_sh.at[pl.ds(r0, stripe)],
                        out_hbm.at[pl.ds(cid * npad + r0, stripe)])

    return k


def _sc_scatter(ne, ridx, npad):
    epad = ne.shape[0]
    z = jnp.zeros((npad // 16, D), F32)
    out = _scatter_kernel(npad, epad)(ne, ridx, z)
    return out.reshape(2, npad, D)


def _dot(a, b):
    return jnp.dot(a.astype(jnp.bfloat16), b.astype(jnp.bfloat16),
                   preferred_element_type=F32)


def _ln(h, s, t):
    mu = jnp.mean(h, axis=-1, keepdims=True)
    var = jnp.mean(jnp.square(h - mu), axis=-1, keepdims=True)
    return (h - mu) / jnp.sqrt(var + 1e-5) * s + t


def _row2(i):
    return (i, 0)


def _fix2(i):
    return (0, 0)


def _edge_mlp(g1, g2, el, w1e, b1, w2, b2, w3, b3, lns, lnb):
    epad = el.shape[0]

    def body(g1_r, g2_r, el_r, w1_r, b1_r, w2_r, b2_r, w3_r, b3_r, s_r, t_r,
             ne_r, eo_r):
        el_v = el_r[...]
        h = g1_r[...] + g2_r[...] + _dot(el_v, w1_r[...]) + b1_r[...]
        h = jnp.maximum(h, 0.0)
        h = jnp.maximum(_dot(h, w2_r[...]) + b2_r[...], 0.0)
        h = _dot(h, w3_r[...]) + b3_r[...]
        ne = _ln(h, s_r[...], t_r[...])
        ne_r[...] = ne
        eo_r[...] = el_v + ne

    return pl.pallas_call(
        body,
        grid=(epad // TE,),
        in_specs=[pl.BlockSpec((TE, D), _row2)] * 3 + [
            pl.BlockSpec((D, D), _fix2), pl.BlockSpec((1, D), _fix2),
            pl.BlockSpec((D, D), _fix2), pl.BlockSpec((1, D), _fix2),
            pl.BlockSpec((D, D), _fix2), pl.BlockSpec((1, D), _fix2),
            pl.BlockSpec((1, D), _fix2), pl.BlockSpec((1, D), _fix2),
        ],
        out_specs=[pl.BlockSpec((TE, D), _row2)] * 2,
        out_shape=[jax.ShapeDtypeStruct((epad, D), F32)] * 2,
    )(g1, g2, el, w1e, b1.reshape(1, D), w2, b2.reshape(1, D), w3,
      b3.reshape(1, D), lns.reshape(1, D), lnb.reshape(1, D))


def _node_mlp(nl, agg, w1n, w1a, b1, w2, b2, w3, b3, lns, lnb, wps, wpr, tn):
    n = nl.shape[0]
    npad = agg.shape[1]

    def body(nl_r, a0_r, a1_r, w1n_r, w1a_r, b1_r, w2_r, b2_r, w3_r, b3_r,
             s_r, t_r, wps_r, wpr_r, no_r, p1_r, p2_r):
        nl_v = nl_r[...]
        a = a0_r[0] + a1_r[0]
        h = _dot(nl_v, w1n_r[...]) + _dot(a, w1a_r[...]) + b1_r[...]
        h = jnp.maximum(h, 0.0)
        h = jnp.maximum(_dot(h, w2_r[...]) + b2_r[...], 0.0)
        h = _dot(h, w3_r[...]) + b3_r[...]
        nn = _ln(h, s_r[...], t_r[...])
        nln = nl_v + nn
        no_r[...] = nln
        p1_r[...] = _dot(nln, wps_r[...])
        p2_r[...] = _dot(nln, wpr_r[...])

    a0m = lambda i: (0, i, 0)
    a1m = lambda i: (1, i, 0)
    return pl.pallas_call(
        body,
        grid=(n // tn,),
        in_specs=[
            pl.BlockSpec((tn, D), _row2),
            pl.BlockSpec((1, tn, D), a0m),
            pl.BlockSpec((1, tn, D), a1m),
            pl.BlockSpec((D, D), _fix2), pl.BlockSpec((D, D), _fix2),
            pl.BlockSpec((1, D), _fix2),
            pl.BlockSpec((D, D), _fix2), pl.BlockSpec((1, D), _fix2),
            pl.BlockSpec((D, D), _fix2), pl.BlockSpec((1, D), _fix2),
            pl.BlockSpec((1, D), _fix2), pl.BlockSpec((1, D), _fix2),
            pl.BlockSpec((D, D), _fix2), pl.BlockSpec((D, D), _fix2),
        ],
        out_specs=[pl.BlockSpec((tn, D), _row2)] * 3,
        out_shape=[jax.ShapeDtypeStruct((n, D), F32)] * 3,
    )(nl, agg, agg, w1n, w1a, b1.reshape(1, D), w2, b2.reshape(1, D), w3,
      b3.reshape(1, D), lns.reshape(1, D), lnb.reshape(1, D), wps, wpr)


def _node_encoder(nf, p, wps, wpr, tn):
    n, fw = nf.shape

    def body(x_r, w1_r, b1_r, w2_r, b2_r, w3_r, b3_r, s_r, t_r, wps_r, wpr_r,
             no_r, p1_r, p2_r):
        h = jnp.maximum(_dot(x_r[...], w1_r[...]) + b1_r[...], 0.0)
        h = jnp.maximum(_dot(h, w2_r[...]) + b2_r[...], 0.0)
        h = _dot(h, w3_r[...]) + b3_r[...]
        nl = _ln(h, s_r[...], t_r[...])
        no_r[...] = nl
        p1_r[...] = _dot(nl, wps_r[...])
        p2_r[...] = _dot(nl, wpr_r[...])

    return pl.pallas_call(
        body,
        grid=(n // tn,),
        in_specs=[
            pl.BlockSpec((tn, fw), _row2),
            pl.BlockSpec((fw, D), _fix2), pl.BlockSpec((1, D), _fix2),
            pl.BlockSpec((D, D), _fix2), pl.BlockSpec((1, D), _fix2),
            pl.BlockSpec((D, D), _fix2), pl.BlockSpec((1, D), _fix2),
            pl.BlockSpec((1, D), _fix2), pl.BlockSpec((1, D), _fix2),
            pl.BlockSpec((D, D), _fix2), pl.BlockSpec((D, D), _fix2),
        ],
        out_specs=[pl.BlockSpec((tn, D), _row2)] * 3,
        out_shape=[jax.ShapeDtypeStruct((n, D), F32)] * 3,
    )(nf, p['w1'], p['b1'].reshape(1, D), p['w2'], p['b2'].reshape(1, D),
      p['w3'], p['b3'].reshape(1, D), p['ln_s'].reshape(1, D),
      p['ln_b'].reshape(1, D), wps, wpr)


def _edge_encoder(ef, p):
    epad, fw = ef.shape

    def body(x_r, w1_r, b1_r, w2_r, b2_r, w3_r, b3_r, s_r, t_r, eo_r):
        h = jnp.maximum(_dot(x_r[...], w1_r[...]) + b1_r[...], 0.0)
        h = jnp.maximum(_dot(h, w2_r[...]) + b2_r[...], 0.0)
        h = _dot(h, w3_r[...]) + b3_r[...]
        eo_r[...] = _ln(h, s_r[...], t_r[...])

    return pl.pallas_call(
        body,
        grid=(epad // TE,),
        in_specs=[
            pl.BlockSpec((TE, fw), _row2),
            pl.BlockSpec((fw, D), _fix2), pl.BlockSpec((1, D), _fix2),
            pl.BlockSpec((D, D), _fix2), pl.BlockSpec((1, D), _fix2),
            pl.BlockSpec((D, D), _fix2), pl.BlockSpec((1, D), _fix2),
            pl.BlockSpec((1, D), _fix2), pl.BlockSpec((1, D), _fix2),
        ],
        out_specs=pl.BlockSpec((TE, D), _row2),
        out_shape=jax.ShapeDtypeStruct((epad, D), F32),
    )(ef, p['w1'], p['b1'].reshape(1, D), p['w2'], p['b2'].reshape(1, D),
      p['w3'], p['b3'].reshape(1, D), p['ln_s'].reshape(1, D),
      p['ln_b'].reshape(1, D))


def _decoder(nl, p, tn):
    n = nl.shape[0]
    ow = p['w3'].shape[1]

    def body(x_r, w1_r, b1_r, w2_r, b2_r, w3_r, b3_r, o_r):
        h = jnp.maximum(_dot(x_r[...], w1_r[...]) + b1_r[...], 0.0)
        h = jnp.maximum(_dot(h, w2_r[...]) + b2_r[...], 0.0)
        o_r[...] = _dot(h, w3_r[...]) + b3_r[...]

    return pl.pallas_call(
        body,
        grid=(n // tn,),
        in_specs=[
            pl.BlockSpec((tn, D), _row2),
            pl.BlockSpec((D, D), _fix2), pl.BlockSpec((1, D), _fix2),
            pl.BlockSpec((D, D), _fix2), pl.BlockSpec((1, D), _fix2),
            pl.BlockSpec((D, ow), _fix2), pl.BlockSpec((1, ow), _fix2),
        ],
        out_specs=pl.BlockSpec((tn, ow), _row2),
        out_shape=jax.ShapeDtypeStruct((n, ow), F32),
    )(nl, p['w1'], p['b1'].reshape(1, D), p['w2'], p['b2'].reshape(1, D),
      p['w3'], p['b3'].reshape(1, ow))


def kernel(world_pos, prev_world_pos, mesh_pos, node_type, cells, params,
           is_training):
    n = world_pos.shape[0]
    tn = 1000 if n % 1000 == 0 else n
    cap = 3 * cells.shape[0]
    e = 2 * cap
    epad = _cdiv(e, NW * CH) * NW * CH
    npad = _cdiv(n + 1, 16 * 8) * 16 * 8

    edges = jnp.concatenate(
        [cells[:, 0:2], cells[:, 1:3],
         jnp.stack([cells[:, 2], cells[:, 0]], axis=1)], axis=0)
    r = jnp.min(edges, axis=1)
    s = jnp.max(edges, axis=1)
    packed = s * n + r
    uniq = jnp.unique(packed, size=cap, fill_value=-1)
    valid = uniq >= 0
    us = jnp.where(valid, uniq // n, 0)
    ur = jnp.where(valid, uniq % n, 0)
    pad = epad - e
    zpad = jnp.zeros((pad,), us.dtype)
    sidx = jnp.concatenate([us, ur, zpad]).astype(jnp.int32)
    ridx = jnp.concatenate([ur, us, zpad]).astype(jnp.int32)
    maskp = jnp.concatenate([valid, valid, jnp.zeros((pad,), jnp.bool_)])
    ridx_m = jnp.where(maskp, ridx, n).astype(jnp.int32)

    velocity = world_pos - prev_world_pos
    one_hot = jax.nn.one_hot(node_type[:, 0], NTYPE, dtype=F32)
    nf = jnp.concatenate([velocity, one_hot], axis=-1)
    nf_mean = jnp.mean(nf, axis=0)
    nf_sq = jnp.mean(nf * nf, axis=0)
    nf_std = jnp.maximum(jnp.sqrt(jnp.maximum(nf_sq - nf_mean * nf_mean, 0.0)),
                         1e-8)
    nf_norm = (nf - nf_mean) / nf_std

    pos = jnp.concatenate(
        [world_pos, mesh_pos, jnp.zeros((n, D - 5), F32)], axis=-1)
    gs, gr = _sc_gather2(pos, pos, sidx, ridx)
    rel = gs - gr
    rel_w = rel[:, 0:3]
    rel_m = rel[:, 3:5]
    ef = jnp.concatenate([
        rel_w, jnp.linalg.norm(rel_w, axis=-1, keepdims=True),
        rel_m, jnp.linalg.norm(rel_m, axis=-1, keepdims=True)], axis=-1)
    mf = maskp.astype(F32)[:, None]
    cnt = jnp.sum(mf)
    ef_mean = jnp.sum(ef * mf, axis=0) / cnt
    ef_sq = jnp.sum(ef * ef * mf, axis=0) / cnt
    ef_std = jnp.maximum(jnp.sqrt(jnp.maximum(ef_sq - ef_mean * ef_mean, 0.0)),
                         1e-8)
    ef_norm = (ef - ef_mean) / ef_std

    blocks = params['blocks']
    w1e0 = blocks[0]['edge']['w1']
    nl, p1, p2 = _node_encoder(nf_norm, params['node_enc'],
                               w1e0[0:D], w1e0[D:2 * D], tn)
    el = _edge_encoder(ef_norm, params['edge_enc'])

    for i, blk in enumerate(blocks):
        g1, g2 = _sc_gather2(p1, p2, sidx, ridx)
        ep = blk['edge']
        ne, el = _edge_mlp(g1, g2, el, ep['w1'][2 * D:], ep['b1'], ep['w2'],
                           ep['b2'], ep['w3'], ep['b3'], ep['ln_s'],
                           ep['ln_b'])
        agg = _sc_scatter(ne, ridx_m, npad)
        npr = blk['node']
        if i + 1 < len(blocks):
            w1n = blocks[i + 1]['edge']['w1']
            wps, wpr = w1n[0:D], w1n[D:2 * D]
        else:
            wps = wpr = jnp.zeros((D, D), F32)
        nl, p1, p2 = _node_mlp(nl, agg, npr['w1'][:D], npr['w1'][D:],
                               npr['b1'], npr['w2'], npr['b2'], npr['w3'],
                               npr['b3'], npr['ln_s'], npr['ln_b'],
                               wps, wpr, tn)

    out = _decoder(nl, params['decoder'], tn)
    integrated = 2 * world_pos + out - prev_world_pos
    return jnp.where(is_training != 0, out, integrated)
